# Initial kernel scaffold; baseline (speedup 1.0000x reference)
import jax
import jax.numpy as jnp
from jax.experimental import pallas as pl


def kernel(x, basis_param, grid, training):
    raise NotImplementedError("write your pallas kernel here")


# speedup vs baseline: 3.7865x; 3.7865x over previous
import jax
import jax.numpy as jnp
from jax.experimental import pallas as pl
from jax.experimental.pallas import tpu as pltpu

_BM = 2048
_N = 8
_J = 256


def _vq_block_kernel(xT_ref, x_ref, bT_ref, naug_ref, gt_ref, m2_ref,
                     xhat_ref, dist_ref):
    xt = xT_ref[...]
    db_t = jnp.zeros_like(xt)
    for i in range(_N - 1, -1, -1):
        bii = bT_ref[i:i + 1, i:i + 1]
        ui = jnp.floor(xt[i:i + 1, :] / bii)
        col = bT_ref[:, i:i + 1]
        upd = col * ui
        xt = xt - upd
        db_t = db_t + upd
    d_b = db_t.T
    x = x_ref[...]
    xb = x.astype(jnp.bfloat16).astype(jnp.float32)
    nx = jnp.sum(x * x, axis=1, keepdims=True)
    ndb = jnp.sum(d_b * d_b, axis=1, keepdims=True)
    sdb = jnp.sum(d_b * xb, axis=1, keepdims=True)
    naug_l = jnp.concatenate(
        [d_b, ndb, jnp.ones_like(ndb),
         jnp.zeros((d_b.shape[0], 6), jnp.float32)], axis=1)
    normc = jax.lax.dot_general(
        naug_l, naug_ref[...], (((1,), (0,)), ((), ())),
        preferred_element_type=jnp.float32,
        precision=jax.lax.Precision.HIGHEST)
    t = jax.lax.dot_general(
        xb, gt_ref[...], (((1,), (0,)), ((), ())),
        preferred_element_type=jnp.float32,
        precision=jax.lax.Precision.HIGHEST)
    e = sdb + t
    dist = ((nx + normc) - (e + e)) * 0.125
    dist_ref[...] = dist
    m = jnp.min(dist, axis=1, keepdims=True)
    iota = jax.lax.broadcasted_iota(jnp.int32, dist.shape, 1)
    idx = jnp.min(jnp.where(dist == m, iota, _J), axis=1,
                  keepdims=True)
    shifts = jax.lax.broadcasted_iota(jnp.int32, (1, _N), 1)
    bits = jnp.right_shift(idx, (_N - 1) - shifts) & 1
    aug2 = jnp.concatenate([bits.astype(jnp.float32), d_b], axis=1)
    xhat_ref[...] = jax.lax.dot_general(
        aug2, m2_ref[...], (((1,), (0,)), ((), ())),
        preferred_element_type=jnp.float32,
        precision=jax.lax.Precision.HIGHEST)


def kernel(x, basis_param, grid, training):
    del training
    B = x.shape[0]
    basis = jnp.tril(basis_param)
    basis = basis / jnp.abs(jnp.linalg.det(basis))
    G = grid @ basis
    naug = jnp.zeros((16, _J), jnp.float32)
    naug = naug.at[0:_N, :].set(G.T * 2.0)
    naug = naug.at[_N, :].set(1.0)
    naug = naug.at[_N + 1, :].set(jnp.sum(G * G, axis=1))
    m2 = jnp.concatenate([basis, jnp.eye(_N, dtype=jnp.float32)], axis=0)
    xT = x.T

    grid_steps = B // _BM
    xhat, dist = pl.pallas_call(
        _vq_block_kernel,
        grid=(grid_steps,),
        in_specs=[
            pl.BlockSpec((_N, _BM), lambda i: (0, i)),
            pl.BlockSpec((_BM, _N), lambda i: (i, 0)),
            pl.BlockSpec((_N, _N), lambda i: (0, 0)),
            pl.BlockSpec((16, _J), lambda i: (0, 0)),
            pl.BlockSpec((_N, _J), lambda i: (0, 0)),
            pl.BlockSpec((16, _N), lambda i: (0, 0)),
        ],
        out_specs=[
            pl.BlockSpec((_BM, _N), lambda i: (i, 0)),
            pl.BlockSpec((_BM, _J), lambda i: (i, 0)),
        ],
        out_shape=[
            jax.ShapeDtypeStruct((B, _N), jnp.float32),
            jax.ShapeDtypeStruct((B, _J), jnp.float32),
        ],
        compiler_params=pltpu.CompilerParams(
            dimension_semantics=("parallel",)),
    )(xT, x, basis.T, naug, G.T, m2)
    return (xhat, xhat, dist)


# speedup vs baseline: 3.9391x; 1.0403x over previous
import jax
import jax.numpy as jnp
from jax.experimental import pallas as pl
from jax.experimental.pallas import tpu as pltpu

_BM = 2048
_N = 8
_J = 256


def _vq_block_kernel(xT_ref, x_ref, bT_ref, g2t_ref, gt_ref, ng_ref,
                     r24_ref, m2_ref, xhat_ref, dist_ref):
    xt = xT_ref[...]
    db_t = jnp.zeros_like(xt)
    for i in range(_N - 1, -1, -1):
        bii = bT_ref[i:i + 1, i:i + 1]
        ui = jnp.floor(xt[i:i + 1, :] / bii)
        col = bT_ref[:, i:i + 1]
        upd = col * ui
        xt = xt - upd
        db_t = db_t + upd
    d_b = db_t.T
    x = x_ref[...]
    xb16 = x.astype(jnp.bfloat16)
    xb = xb16.astype(jnp.float32)
    l24 = jnp.concatenate([x * x, d_b * d_b, d_b * xb], axis=1)
    s = jax.lax.dot_general(
        l24, r24_ref[...], (((1,), (0,)), ((), ())),
        preferred_element_type=jnp.float32,
        precision=jax.lax.Precision.HIGHEST)
    nx = s[:, 0:1]
    ndb = s[:, 1:2]
    sdb = s[:, 2:3]
    p = jax.lax.dot_general(
        d_b.astype(jnp.bfloat16), g2t_ref[...], (((1,), (0,)), ((), ())),
        preferred_element_type=jnp.float32)
    t = jax.lax.dot_general(
        xb16, gt_ref[...], (((1,), (0,)), ((), ())),
        preferred_element_type=jnp.float32)
    z = ndb + ng_ref[0:1, :]
    normc = p + z
    e = sdb + t
    dist = ((nx + normc) - (e + e)) * 0.125
    dist_ref[...] = dist
    m = jnp.min(dist, axis=1, keepdims=True)
    iota = jax.lax.broadcasted_iota(jnp.int32, dist.shape, 1)
    idx = jnp.min(jnp.where(dist == m, iota, _J), axis=1,
                  keepdims=True)
    shifts = jax.lax.broadcasted_iota(jnp.int32, (1, _N), 1)
    bits = jnp.right_shift(idx, (_N - 1) - shifts) & 1
    aug2 = jnp.concatenate(
        [bits.astype(jnp.bfloat16), d_b.astype(jnp.bfloat16)], axis=1)
    xhat_ref[...] = jax.lax.dot_general(
        aug2, m2_ref[...], (((1,), (0,)), ((), ())),
        preferred_element_type=jnp.float32)


def kernel(x, basis_param, grid, training):
    del training
    B = x.shape[0]
    basis = jnp.tril(basis_param)
    basis = basis / jnp.abs(jnp.linalg.det(basis))
    G = grid @ basis
    g2t = (2.0 * G.T).astype(jnp.bfloat16)
    gt = G.T.astype(jnp.bfloat16)
    ng = jnp.zeros((_N, _J), jnp.float32).at[0, :].set(jnp.sum(G * G, axis=1))
    r24 = jnp.zeros((24, _N), jnp.float32)
    r24 = r24.at[0:8, 0].set(1.0).at[8:16, 1].set(1.0).at[16:24, 2].set(1.0)
    m2 = jnp.concatenate(
        [basis, jnp.eye(_N, dtype=jnp.float32)], axis=0).astype(jnp.bfloat16)
    xT = x.T

    grid_steps = B // _BM
    xhat, dist = pl.pallas_call(
        _vq_block_kernel,
        grid=(grid_steps,),
        in_specs=[
            pl.BlockSpec((_N, _BM), lambda i: (0, i)),
            pl.BlockSpec((_BM, _N), lambda i: (i, 0)),
            pl.BlockSpec((_N, _N), lambda i: (0, 0)),
            pl.BlockSpec((_N, _J), lambda i: (0, 0)),
            pl.BlockSpec((_N, _J), lambda i: (0, 0)),
            pl.BlockSpec((_N, _J), lambda i: (0, 0)),
            pl.BlockSpec((24, _N), lambda i: (0, 0)),
            pl.BlockSpec((16, _N), lambda i: (0, 0)),
        ],
        out_specs=[
            pl.BlockSpec((_BM, _N), lambda i: (i, 0)),
            pl.BlockSpec((_BM, _J), lambda i: (i, 0)),
        ],
        out_shape=[
            jax.ShapeDtypeStruct((B, _N), jnp.float32),
            jax.ShapeDtypeStruct((B, _J), jnp.float32),
        ],
        compiler_params=pltpu.CompilerParams(
            dimension_semantics=("parallel",)),
    )(xT, x, basis.T, g2t, gt, ng, r24, m2)
    return (xhat, xhat, dist)


# speedup vs baseline: 5.4329x; 1.3792x over previous
import jax
import jax.numpy as jnp
from jax.experimental import pallas as pl
from jax.experimental.pallas import tpu as pltpu

_BM = 8192
_N = 8
_J = 256


def _vq_block_kernel(xT_ref, x_ref, bT_ref, g2t_ref, gt_ref, ng_ref,
                     r24_ref, m2_ref, xhat_ref, dist_ref):
    xt = xT_ref[...]
    db_t = jnp.zeros_like(xt)
    for i in range(_N - 1, -1, -1):
        bii = bT_ref[i:i + 1, i:i + 1]
        ui = jnp.floor(xt[i:i + 1, :] / bii)
        col = bT_ref[:, i:i + 1]
        upd = col * ui
        xt = xt - upd
        db_t = db_t + upd
    d_b = db_t.T
    x = x_ref[...]
    xb16 = x.astype(jnp.bfloat16)
    xb = xb16.astype(jnp.float32)
    l24 = jnp.concatenate([x * x, d_b * d_b, d_b * xb], axis=1)
    s = jax.lax.dot_general(
        l24, r24_ref[...], (((1,), (0,)), ((), ())),
        preferred_element_type=jnp.float32,
        precision=jax.lax.Precision.HIGHEST)
    nx = s[:, 0:1]
    ndb = s[:, 1:2]
    sdb = s[:, 2:3]
    p = jax.lax.dot_general(
        d_b.astype(jnp.bfloat16), g2t_ref[...], (((1,), (0,)), ((), ())),
        preferred_element_type=jnp.float32)
    t = jax.lax.dot_general(
        xb16, gt_ref[...], (((1,), (0,)), ((), ())),
        preferred_element_type=jnp.float32)
    z = ndb + ng_ref[0:1, :]
    normc = p + z
    e = sdb + t
    dist = ((nx + normc) - (e + e)) * 0.125
    dist_ref[...] = dist
    m = jnp.min(dist, axis=1, keepdims=True)
    iota = jax.lax.broadcasted_iota(jnp.int32, dist.shape, 1)
    idx = jnp.min(jnp.where(dist == m, iota, _J), axis=1,
                  keepdims=True)
    shifts = jax.lax.broadcasted_iota(jnp.int32, (1, _N), 1)
    bits = jnp.right_shift(idx, (_N - 1) - shifts) & 1
    aug2 = jnp.concatenate(
        [bits.astype(jnp.bfloat16), d_b.astype(jnp.bfloat16)], axis=1)
    xhat_ref[...] = jax.lax.dot_general(
        aug2, m2_ref[...], (((1,), (0,)), ((), ())),
        preferred_element_type=jnp.float32)


def kernel(x, basis_param, grid, training):
    del training
    B = x.shape[0]
    basis = jnp.tril(basis_param)
    basis = basis / jnp.abs(jnp.linalg.det(basis))
    G = grid @ basis
    g2t = (2.0 * G.T).astype(jnp.bfloat16)
    gt = G.T.astype(jnp.bfloat16)
    ng = jnp.zeros((_N, _J), jnp.float32).at[0, :].set(jnp.sum(G * G, axis=1))
    r24 = jnp.zeros((24, _N), jnp.float32)
    r24 = r24.at[0:8, 0].set(1.0).at[8:16, 1].set(1.0).at[16:24, 2].set(1.0)
    m2 = jnp.concatenate(
        [basis, jnp.eye(_N, dtype=jnp.float32)], axis=0).astype(jnp.bfloat16)
    xT = x.T

    grid_steps = B // _BM
    xhat, dist = pl.pallas_call(
        _vq_block_kernel,
        grid=(grid_steps,),
        in_specs=[
            pl.BlockSpec((_N, _BM), lambda i: (0, i)),
            pl.BlockSpec((_BM, _N), lambda i: (i, 0)),
            pl.BlockSpec((_N, _N), lambda i: (0, 0)),
            pl.BlockSpec((_N, _J), lambda i: (0, 0)),
            pl.BlockSpec((_N, _J), lambda i: (0, 0)),
            pl.BlockSpec((_N, _J), lambda i: (0, 0)),
            pl.BlockSpec((24, _N), lambda i: (0, 0)),
            pl.BlockSpec((16, _N), lambda i: (0, 0)),
        ],
        out_specs=[
            pl.BlockSpec((_BM, _N), lambda i: (i, 0)),
            pl.BlockSpec((_BM, _J), lambda i: (i, 0)),
        ],
        out_shape=[
            jax.ShapeDtypeStruct((B, _N), jnp.float32),
            jax.ShapeDtypeStruct((B, _J), jnp.float32),
        ],
        compiler_params=pltpu.CompilerParams(
            dimension_semantics=("parallel",)),
    )(xT, x, basis.T, g2t, gt, ng, r24, m2)
    return (xhat, xhat, dist)


# speedup vs baseline: 7.9324x; 1.4601x over previous
import jax
import jax.numpy as jnp
from jax.experimental import pallas as pl
from jax.experimental.pallas import tpu as pltpu

_BM = 8192
_N = 8
_J = 256


def _vq_block_kernel(xT_ref, bT_ref, g2tn_ref, gt_ref, m2_ref,
                     xhat_ref, dist_ref):
    xt = xT_ref[...]
    db_t = jnp.zeros_like(xt)
    for i in range(_N - 1, -1, -1):
        bii = bT_ref[i:i + 1, i:i + 1]
        ui = jnp.floor(xt[i:i + 1, :] / bii)
        col = bT_ref[:, i:i + 1]
        upd = col * ui
        xt = xt - upd
        db_t = db_t + upd
    x0 = xT_ref[...]
    xbT16 = x0.astype(jnp.bfloat16)
    xbT = xbT16.astype(jnp.float32)
    nxT = jnp.sum(x0 * x0, axis=0, keepdims=True)
    ndbT = jnp.sum(db_t * db_t, axis=0, keepdims=True)
    sdbT = jnp.sum(db_t * xbT, axis=0, keepdims=True)
    pack = jnp.concatenate(
        [nxT, ndbT, sdbT, jnp.zeros((5, nxT.shape[1]), jnp.float32)],
        axis=0)
    srow = pack.T
    nx = srow[:, 0:1]
    ndb = srow[:, 1:2]
    sdb = srow[:, 2:3]
    db16 = db_t.astype(jnp.bfloat16)
    lhsp = jnp.concatenate(
        [db16, jnp.ones_like(db16[0:1, :]),
         jnp.zeros((7, db16.shape[1]), jnp.bfloat16)], axis=0)
    p = jax.lax.dot_general(
        lhsp, g2tn_ref[...], (((0,), (0,)), ((), ())),
        preferred_element_type=jnp.float32)
    t = jax.lax.dot_general(
        xbT16, gt_ref[...], (((0,), (0,)), ((), ())),
        preferred_element_type=jnp.float32)
    normc = p + ndb
    e = sdb + t
    dist = ((nx + normc) - (e + e)) * 0.125
    dist_ref[...] = dist
    m = jnp.min(dist, axis=1, keepdims=True)
    iota = jax.lax.broadcasted_iota(jnp.int32, dist.shape, 1)
    idx = jnp.min(jnp.where(dist == m, iota, _J), axis=1,
                  keepdims=True)
    ipack = jnp.concatenate(
        [idx, jnp.zeros((idx.shape[0], _N - 1), jnp.int32)], axis=1)
    idxT = ipack.T[0:1, :]
    shifts = jax.lax.broadcasted_iota(jnp.int32, (_N, 1), 0)
    bitsT = jnp.right_shift(idxT, (_N - 1) - shifts) & 1
    aug2T = jnp.concatenate([bitsT.astype(jnp.bfloat16), db16], axis=0)
    xhat_ref[...] = jax.lax.dot_general(
        aug2T, m2_ref[...], (((0,), (0,)), ((), ())),
        preferred_element_type=jnp.float32)


def kernel(x, basis_param, grid, training):
    del training
    B = x.shape[0]
    basis = jnp.tril(basis_param)
    basis = basis / jnp.abs(jnp.linalg.det(basis))
    G = grid @ basis
    g2tn = jnp.zeros((16, _J), jnp.float32)
    g2tn = g2tn.at[0:_N, :].set(2.0 * G.T)
    g2tn = g2tn.at[_N, :].set(jnp.sum(G * G, axis=1))
    g2tn = g2tn.astype(jnp.bfloat16)
    gt = G.T.astype(jnp.bfloat16)
    m2 = jnp.concatenate(
        [basis, jnp.eye(_N, dtype=jnp.float32)], axis=0).astype(jnp.bfloat16)
    xT = x.T

    grid_steps = B // _BM
    xhat, dist = pl.pallas_call(
        _vq_block_kernel,
        grid=(grid_steps,),
        in_specs=[
            pl.BlockSpec((_N, _BM), lambda i: (0, i)),
            pl.BlockSpec((_N, _N), lambda i: (0, 0)),
            pl.BlockSpec((16, _J), lambda i: (0, 0)),
            pl.BlockSpec((_N, _J), lambda i: (0, 0)),
            pl.BlockSpec((16, _N), lambda i: (0, 0)),
        ],
        out_specs=[
            pl.BlockSpec((_BM, _N), lambda i: (i, 0)),
            pl.BlockSpec((_BM, _J), lambda i: (i, 0)),
        ],
        out_shape=[
            jax.ShapeDtypeStruct((B, _N), jnp.float32),
            jax.ShapeDtypeStruct((B, _J), jnp.float32),
        ],
        compiler_params=pltpu.CompilerParams(
            dimension_semantics=("parallel",)),
    )(xT, basis.T, g2tn, gt, m2)
    return (xhat, xhat, dist)


# speedup vs baseline: 8.6407x; 1.0893x over previous
import jax
import jax.numpy as jnp
from jax.experimental import pallas as pl
from jax.experimental.pallas import tpu as pltpu

_BM = 8192
_N = 8
_J = 256


def _vq_block_kernel(xT_ref, bT_ref, g2tn_ref, gt_ref, m2_ref,
                     xhat_ref, dist_ref):
    xt = xT_ref[...]
    db_t = jnp.zeros_like(xt)
    for i in range(_N - 1, -1, -1):
        bii = bT_ref[i:i + 1, i:i + 1]
        ui = jnp.floor(xt[i:i + 1, :] / bii)
        col = bT_ref[:, i:i + 1]
        upd = col * ui
        xt = xt - upd
        db_t = db_t + upd
    x0 = xT_ref[...]
    xbT16 = x0.astype(jnp.bfloat16)
    xbT = xbT16.astype(jnp.float32)
    nxT = jnp.sum(x0 * x0, axis=0, keepdims=True)
    ndbT = jnp.sum(db_t * db_t, axis=0, keepdims=True)
    sdbT = jnp.sum(db_t * xbT, axis=0, keepdims=True)
    pack = jnp.concatenate(
        [nxT, ndbT, sdbT, jnp.zeros((5, nxT.shape[1]), jnp.float32)],
        axis=0)
    srow = pack.T
    nx = srow[:, 0:1]
    ndb = srow[:, 1:2]
    sdb = srow[:, 2:3]
    db16 = db_t.astype(jnp.bfloat16)
    lhsp = jnp.concatenate(
        [db16, jnp.ones_like(db16[0:1, :]),
         jnp.zeros((7, db16.shape[1]), jnp.bfloat16)], axis=0)
    p = jax.lax.dot_general(
        lhsp, g2tn_ref[...], (((0,), (0,)), ((), ())),
        preferred_element_type=jnp.float32)
    t = jax.lax.dot_general(
        xbT16, gt_ref[...], (((0,), (0,)), ((), ())),
        preferred_element_type=jnp.float32)
    normc = p + ndb
    e = sdb + t
    dist = ((nx + normc) - (e + e)) * 0.125
    dist_ref[...] = dist
    m = jnp.min(dist, axis=1, keepdims=True)
    iota = jax.lax.broadcasted_iota(jnp.int32, dist.shape, 1)
    idx = jnp.min(jnp.where(dist == m, iota, _J), axis=1,
                  keepdims=True)
    ipack = jnp.concatenate(
        [idx, jnp.zeros((idx.shape[0], _N - 1), jnp.int32)], axis=1)
    idxT = ipack.T[0:1, :]
    shifts = jax.lax.broadcasted_iota(jnp.int32, (_N, 1), 0)
    bitsT = jnp.right_shift(idxT, (_N - 1) - shifts) & 1
    aug2T = jnp.concatenate([bitsT.astype(jnp.bfloat16), db16], axis=0)
    xhat_ref[...] = jax.lax.dot_general(
        aug2T, m2_ref[...], (((0,), (0,)), ((), ())),
        preferred_element_type=jnp.float32)


def kernel(x, basis_param, grid, training):
    del training
    B = x.shape[0]
    basis = jnp.tril(basis_param)
    basis = basis / jnp.abs(jnp.prod(jnp.diag(basis)))
    G = grid @ basis
    g2tn = jnp.zeros((16, _J), jnp.float32)
    g2tn = g2tn.at[0:_N, :].set(2.0 * G.T)
    g2tn = g2tn.at[_N, :].set(jnp.sum(G * G, axis=1))
    g2tn = g2tn.astype(jnp.bfloat16)
    gt = G.T.astype(jnp.bfloat16)
    m2 = jnp.concatenate(
        [basis, jnp.eye(_N, dtype=jnp.float32)], axis=0).astype(jnp.bfloat16)
    xT = x.T

    grid_steps = B // _BM
    xhat, dist = pl.pallas_call(
        _vq_block_kernel,
        grid=(grid_steps,),
        in_specs=[
            pl.BlockSpec((_N, _BM), lambda i: (0, i)),
            pl.BlockSpec((_N, _N), lambda i: (0, 0)),
            pl.BlockSpec((16, _J), lambda i: (0, 0)),
            pl.BlockSpec((_N, _J), lambda i: (0, 0)),
            pl.BlockSpec((16, _N), lambda i: (0, 0)),
        ],
        out_specs=[
            pl.BlockSpec((_BM, _N), lambda i: (i, 0)),
            pl.BlockSpec((_BM, _J), lambda i: (i, 0)),
        ],
        out_shape=[
            jax.ShapeDtypeStruct((B, _N), jnp.float32),
            jax.ShapeDtypeStruct((B, _J), jnp.float32),
        ],
        compiler_params=pltpu.CompilerParams(
            dimension_semantics=("parallel",)),
    )(xT, basis.T, g2tn, gt, m2)
    return (xhat, xhat, dist)
